# baseline (speedup 1.0000x reference)
import functools

import jax
import jax.numpy as jnp
from jax import lax
from jax.experimental import pallas as pl
from jax.experimental.pallas import tpu as pltpu
from jax.experimental.pallas import tpu_sc as plsc

NC, NS = 2, 16
SUB = 128
CW = 16
NCHUNK = 8
ZROWS = 256
NBUF = 8


def _sc_segment_sums(feat_packet, feat_router, ep_s, ep_d, ec_s, ec_d,
                     et_s, et_d, n, h):
    e_pad = ep_s.shape[0] * ep_s.shape[1]
    spt = e_pad // (SUB * NS)
    n_pad = ((n + 1 + 8 * NS - 1) // (8 * NS)) * (8 * NS)
    rpt = n_pad // NS

    fp_flat = feat_packet.reshape(n * NCHUNK, CW)
    fr_flat = feat_router.reshape(n * NCHUNK, CW)

    z32 = jnp.zeros((ZROWS, CW), jnp.float32)
    ones_in = jnp.ones((SUB, CW), jnp.float32)

    mesh = plsc.VectorSubcoreMesh(core_axis_name="c", subcore_axis_name="s")
    out_sds = jax.ShapeDtypeStruct((NCHUNK, n_pad, CW), jnp.float32)

    @functools.partial(
        pl.kernel,
        out_type=(out_sds, out_sds, out_sds, out_sds),
        mesh=mesh,
        compiler_params=pltpu.CompilerParams(use_tc_tiling_on_sc=False),
        scratch_types=dict(
            srcb=pltpu.VMEM((spt, SUB), jnp.int32),
            dstb=pltpu.VMEM((spt, SUB), jnp.int32),
            rows=pltpu.VMEM((NBUF, SUB, CW), jnp.float32),
            onesb=pltpu.VMEM((SUB, CW), jnp.float32),
            acc=pltpu.VMEM_SHARED((n_pad, CW), jnp.float32),
            sems=(pltpu.SemaphoreType.DMA,) * NBUF,
        ),
    )
    def sc_kernel(fp_hbm, fr_hbm, eps_h, epd_h, ecs_h, ecd_h, ets_h, etd_h,
                  z32_h, ones_h,
                  h1_out, h2_out, sq_out, dg_out,
                  srcb, dstb, rows, onesb, acc, sems):
        c = lax.axis_index("c")
        s = lax.axis_index("s")

        pltpu.sync_copy(ones_h, onesb)

        def zero_acc():
            nfull = rpt // ZROWS
            for k in range(nfull):
                pltpu.sync_copy(z32_h,
                                acc.at[pl.ds(s * rpt + k * ZROWS, ZROWS)])
            rem = rpt - nfull * ZROWS
            if rem:
                pltpu.sync_copy(z32_h.at[pl.ds(0, rem)],
                                acc.at[pl.ds(s * rpt + nfull * ZROWS, rem)])

        relations = (
            (fp_hbm, eps_h, epd_h, h1_out),
            (fr_hbm, ecs_h, ecd_h, h2_out),
            (fr_hbm, ets_h, etd_h, sq_out),
        )
        for tbl, src_h, dst_h, out_h in relations:
            for p in range(NCHUNK // NC):
                ch = c + NC * p

                zero_acc()
                plsc.subcore_barrier()

                pltpu.sync_copy(src_h.at[pl.ds(s * spt, spt)], srcb)
                pltpu.sync_copy(dst_h.at[pl.ds(s * spt, spt)], dstb)

                def idx_body(r, _, *, _ch=ch):
                    for j in range(SUB // 16):
                        sl = pl.ds(j * 16, 16)
                        srcb[r, sl] = srcb[r, sl] * NCHUNK + _ch
                    return 0
                lax.fori_loop(0, spt, idx_body, 0)

                for k in range(NBUF):
                    pltpu.async_copy(tbl.at[srcb.at[k]], rows.at[k], sems[k])

                def gs_body(g, _, *, _tbl=tbl):
                    for k in range(NBUF):
                        i = NBUF * g + k
                        pltpu.make_async_copy(_tbl.at[srcb.at[i]],
                                              rows.at[k], sems[k]).wait()
                        pltpu.sync_copy(rows.at[k], acc.at[dstb.at[i]],
                                        add=True)

                        @pl.when(i + NBUF < spt)
                        def _start_next():
                            pltpu.async_copy(_tbl.at[srcb.at[i + NBUF]],
                                             rows.at[k], sems[k])
                    return 0
                lax.fori_loop(0, spt // NBUF, gs_body, 0)
                plsc.subcore_barrier()

                pltpu.sync_copy(
                    acc.at[pl.ds(s * rpt, rpt)],
                    out_h.at[ch, pl.ds(s * rpt, rpt)])

        zero_acc()
        plsc.subcore_barrier()
        pltpu.sync_copy(etd_h.at[pl.ds(s * spt, spt)], dstb)

        def deg_body(i, _):
            pltpu.sync_copy(onesb, acc.at[dstb.at[i]], add=True)
            return 0
        lax.fori_loop(0, spt, deg_body, 0)
        plsc.subcore_barrier()

        @pl.when(c == 0)
        def _deg_out():
            pltpu.sync_copy(acc.at[pl.ds(s * rpt, rpt)],
                            dg_out.at[0, pl.ds(s * rpt, rpt)])

    return sc_kernel(fp_flat, fr_flat, ep_s, ep_d, ec_s, ec_d, et_s, et_d,
                     z32, ones_in)


def _mlp_body(h1, h2, fr, wr, br, sq, dg, fp, wp, bp, out_r, out_p):
    z = (jnp.dot(h1[...], wr[...][:128, :], preferred_element_type=jnp.float32,
                 precision=lax.Precision.HIGHEST)
         + jnp.dot(h2[...], wr[...][128:, :], preferred_element_type=jnp.float32,
                   precision=lax.Precision.HIGHEST)
         + br[...])
    out_r[...] = fr[...] + jnp.maximum(z, 0.0)
    d = jnp.maximum(dg[...][:, :1], 1.0)
    z2 = (jnp.dot(sq[...] / d, wp[...], preferred_element_type=jnp.float32,
                  precision=lax.Precision.HIGHEST)
          + bp[...])
    out_p[...] = fp[...] + jnp.maximum(z2, 0.0)


def _prep_edges(e, e_pad, n, n_pad):
    src = e[0].astype(jnp.int32)
    dst = e[1].astype(jnp.int32)
    pad = e_pad - src.shape[0]
    if pad:
        r = jnp.arange(pad, dtype=jnp.int32)
        src = jnp.concatenate([src, r % n])
        dst = jnp.concatenate([dst, n + r % (n_pad - n)])
    return (src.reshape(e_pad // SUB, SUB),
            dst.reshape(e_pad // SUB, SUB))


def kernel(feat_router, feat_packet, edge_pass, edge_transfer, edge_connect,
           lin_r_weight, lin_r_bias, lin_p_weight, lin_p_bias):
    n, h = feat_router.shape
    assert feat_packet.shape == (n, h) and h == NCHUNK * CW

    grp = SUB * NS * 8
    e = edge_pass.shape[1]
    e_pad = ((e + grp - 1) // grp) * grp
    n_pad = ((n + 1 + 8 * NS - 1) // (8 * NS)) * (8 * NS)

    ep_s, ep_d = _prep_edges(edge_pass, e_pad, n, n_pad)
    et_s, et_d = _prep_edges(edge_transfer, e_pad, n, n_pad)
    ec_s, ec_d = _prep_edges(edge_connect, e_pad, n, n_pad)

    h1r, h2r, sq, dg = _sc_segment_sums(
        feat_packet, feat_router, ep_s, ep_d, ec_s, ec_d, et_s, et_d, n, h)

    def _rows(x):
        return jnp.concatenate([x[ch] for ch in range(NCHUNK)], axis=-1)
    h1r, h2r, sqr = _rows(h1r), _rows(h2r), _rows(sq)

    blk = 1000
    grid = (n // blk,)
    row_spec = pl.BlockSpec((blk, h), lambda i: (i, 0))
    d_spec = pl.BlockSpec((blk, CW), lambda i: (i, 0))
    full = lambda shape: pl.BlockSpec(shape, lambda i: (0, 0))

    out_r, out_p = pl.pallas_call(
        _mlp_body,
        grid=grid,
        in_specs=[row_spec, row_spec, row_spec, full((2 * h, h)),
                  full((1, h)), row_spec, d_spec, row_spec,
                  full((h, h)), full((1, h))],
        out_specs=[row_spec, row_spec],
        out_shape=[jax.ShapeDtypeStruct((n, h), jnp.float32),
                   jax.ShapeDtypeStruct((n, h), jnp.float32)],
    )(h1r, h2r, feat_router, lin_r_weight, lin_r_bias.reshape(1, h),
      sqr, dg[0], feat_packet, lin_p_weight, lin_p_bias.reshape(1, h))
    return out_r, out_p


# speedup vs baseline: 1.4801x; 1.4801x over previous
import functools

import jax
import jax.numpy as jnp
from jax import lax
from jax.experimental import pallas as pl
from jax.experimental.pallas import tpu as pltpu
from jax.experimental.pallas import tpu_sc as plsc

NC, NS = 2, 16
SUB = 128
CW = 16
NCHUNK = 8
ZROWS = 256
NBUF = 8


def _sc_segment_sums(feat_packet, feat_router, ep_s, ep_d, ec_s, ec_d,
                     et_s, et_d, n, h):
    e_pad = ep_s.shape[0] * ep_s.shape[1]
    spt = e_pad // (SUB * NS)
    n_pad = ((n + 1 + 8 * NS - 1) // (8 * NS)) * (8 * NS)
    rpt = n_pad // NS

    fp_flat = feat_packet.reshape(n * NCHUNK, CW)
    fr_flat = feat_router.reshape(n * NCHUNK, CW)

    z32 = jnp.zeros((ZROWS, CW), jnp.float32)
    ones_in = jnp.ones((SUB, CW), jnp.float32)

    mesh = plsc.VectorSubcoreMesh(core_axis_name="c", subcore_axis_name="s")
    out_sds = jax.ShapeDtypeStruct((NCHUNK, n_pad, CW), jnp.float32)

    @functools.partial(
        pl.kernel,
        out_type=(out_sds, out_sds, out_sds, out_sds),
        mesh=mesh,
        compiler_params=pltpu.CompilerParams(use_tc_tiling_on_sc=False),
        scratch_types=dict(
            srcb=pltpu.VMEM((spt, SUB), jnp.int32),
            dstb=pltpu.VMEM((spt, SUB), jnp.int32),
            rows=pltpu.VMEM((NBUF, SUB, CW), jnp.float32),
            onesb=pltpu.VMEM((SUB, CW), jnp.float32),
            acc=pltpu.VMEM_SHARED((n_pad, CW), jnp.float32),
            sems=(pltpu.SemaphoreType.DMA,) * NBUF,
        ),
    )
    def sc_kernel(fp_hbm, fr_hbm, eps_h, epd_h, ecs_h, ecd_h, ets_h, etd_h,
                  z32_h, ones_h,
                  h1_out, h2_out, sq_out, dg_out,
                  srcb, dstb, rows, onesb, acc, sems):
        c = lax.axis_index("c")
        s = lax.axis_index("s")

        pltpu.sync_copy(ones_h, onesb)

        def zero_acc():
            nfull = rpt // ZROWS
            for k in range(nfull):
                pltpu.sync_copy(z32_h,
                                acc.at[pl.ds(s * rpt + k * ZROWS, ZROWS)])
            rem = rpt - nfull * ZROWS
            if rem:
                pltpu.sync_copy(z32_h.at[pl.ds(0, rem)],
                                acc.at[pl.ds(s * rpt + nfull * ZROWS, rem)])

        relations = (
            (fp_hbm, eps_h, epd_h, h1_out),
            (fr_hbm, ecs_h, ecd_h, h2_out),
            (fr_hbm, ets_h, etd_h, sq_out),
        )
        for tbl, src_h, dst_h, out_h in relations:
            for p in range(NCHUNK // NC):
                ch = c + NC * p

                zero_acc()
                plsc.subcore_barrier()

                pltpu.sync_copy(src_h.at[pl.ds(s * spt, spt)], srcb)
                pltpu.sync_copy(dst_h.at[pl.ds(s * spt, spt)], dstb)

                def idx_body(r, _, *, _ch=ch):
                    for j in range(SUB // 16):
                        sl = pl.ds(j * 16, 16)
                        srcb[r, sl] = srcb[r, sl] * NCHUNK + _ch
                    return 0
                lax.fori_loop(0, spt, idx_body, 0)

                for k in range(NBUF):
                    pltpu.async_copy(tbl.at[srcb.at[k]], rows.at[k], sems[k])

                def gs_body(g, _, *, _tbl=tbl):
                    for k in range(NBUF):
                        i = NBUF * g + k
                        pltpu.make_async_copy(_tbl.at[srcb.at[i]],
                                              rows.at[k], sems[k]).wait()
                        pltpu.sync_copy(rows.at[k], acc.at[dstb.at[i]],
                                        add=True)

                        @pl.when(i + NBUF < spt)
                        def _start_next():
                            pltpu.async_copy(_tbl.at[srcb.at[i + NBUF]],
                                             rows.at[k], sems[k])
                    return 0
                lax.fori_loop(0, spt // NBUF, gs_body, 0)
                plsc.subcore_barrier()

                pltpu.sync_copy(
                    acc.at[pl.ds(s * rpt, rpt)],
                    out_h.at[ch, pl.ds(s * rpt, rpt)])

        zero_acc()
        plsc.subcore_barrier()
        pltpu.sync_copy(etd_h.at[pl.ds(s * spt, spt)], dstb)

        def deg_body(i, _):
            pltpu.sync_copy(onesb, acc.at[dstb.at[i]], add=True)
            return 0
        lax.fori_loop(0, spt, deg_body, 0)
        plsc.subcore_barrier()

        @pl.when(c == 0)
        def _deg_out():
            pltpu.sync_copy(acc.at[pl.ds(s * rpt, rpt)],
                            dg_out.at[0, pl.ds(s * rpt, rpt)])

    return sc_kernel(fp_flat, fr_flat, ep_s, ep_d, ec_s, ec_d, et_s, et_d,
                     z32, ones_in)


def _mlp_body(h1, h2, fr, wr, br, sq, dg, fp, wp, bp, out_r, out_p):
    z = (jnp.dot(h1[...], wr[...][:128, :], preferred_element_type=jnp.float32,
                 precision=lax.Precision.HIGHEST)
         + jnp.dot(h2[...], wr[...][128:, :], preferred_element_type=jnp.float32,
                   precision=lax.Precision.HIGHEST)
         + br[...])
    out_r[...] = fr[...] + jnp.maximum(z, 0.0)
    d = jnp.maximum(dg[...][:, :1], 1.0)
    z2 = (jnp.dot(sq[...] / d, wp[...], preferred_element_type=jnp.float32,
                  precision=lax.Precision.HIGHEST)
          + bp[...])
    out_p[...] = fp[...] + jnp.maximum(z2, 0.0)


def _prep_edges(e, e_pad, n, n_pad):
    src = e[0].astype(jnp.int32)
    dst = e[1].astype(jnp.int32)
    pad = e_pad - src.shape[0]
    if pad:
        r = jnp.arange(pad, dtype=jnp.int32)
        src = jnp.concatenate([src, r % n])
        dst = jnp.concatenate([dst, n + r % (n_pad - n)])
    return (src.reshape(e_pad // SUB, SUB),
            dst.reshape(e_pad // SUB, SUB))


def kernel(feat_router, feat_packet, edge_pass, edge_transfer, edge_connect,
           lin_r_weight, lin_r_bias, lin_p_weight, lin_p_bias):
    n, h = feat_router.shape
    assert feat_packet.shape == (n, h) and h == NCHUNK * CW

    grp = SUB * NS * 8
    e = edge_pass.shape[1]
    e_pad = ((e + grp - 1) // grp) * grp
    n_pad = ((n + 1 + 8 * NS - 1) // (8 * NS)) * (8 * NS)

    ep_s, ep_d = _prep_edges(edge_pass, e_pad, n, n_pad)
    et_s, et_d = _prep_edges(edge_transfer, e_pad, n, n_pad)
    ec_s, ec_d = _prep_edges(edge_connect, e_pad, n, n_pad)

    h1r, h2r, sq, dg = _sc_segment_sums(
        feat_packet, feat_router, ep_s, ep_d, ec_s, ec_d, et_s, et_d, n, h)

    def _rows(x):
        return x.transpose(1, 0, 2).reshape(x.shape[1], h)
    h1r, h2r, sqr = _rows(h1r), _rows(h2r), _rows(sq)

    blk = 1000
    grid = (n // blk,)
    row_spec = pl.BlockSpec((blk, h), lambda i: (i, 0))
    d_spec = pl.BlockSpec((blk, CW), lambda i: (i, 0))
    full = lambda shape: pl.BlockSpec(shape, lambda i: (0, 0))

    out_r, out_p = pl.pallas_call(
        _mlp_body,
        grid=grid,
        in_specs=[row_spec, row_spec, row_spec, full((2 * h, h)),
                  full((1, h)), row_spec, d_spec, row_spec,
                  full((h, h)), full((1, h))],
        out_specs=[row_spec, row_spec],
        out_shape=[jax.ShapeDtypeStruct((n, h), jnp.float32),
                   jax.ShapeDtypeStruct((n, h), jnp.float32)],
    )(h1r, h2r, feat_router, lin_r_weight, lin_r_bias.reshape(1, h),
      sqr, dg[0], feat_packet, lin_p_weight, lin_p_bias.reshape(1, h))
    return out_r, out_p


# speedup vs baseline: 1.6292x; 1.1007x over previous
import functools

import jax
import jax.numpy as jnp
from jax import lax
from jax.experimental import pallas as pl
from jax.experimental.pallas import tpu as pltpu
from jax.experimental.pallas import tpu_sc as plsc

NC, NS = 2, 16
SUB = 128
CW = 16
NCHUNK = 8
ZROWS = 256
NBUF = 8


def _sc_segment_sums(feat_packet, feat_router, ep_s, ep_d, ec_s, ec_d,
                     et_s, et_d, n, h):
    e_pad = ep_s.shape[0] * ep_s.shape[1]
    spt = e_pad // (SUB * NS)
    n_pad = ((n + 1 + 8 * NS - 1) // (8 * NS)) * (8 * NS)
    rpt = n_pad // NS

    fp_flat = feat_packet.reshape(n * NCHUNK, CW)
    fr_flat = feat_router.reshape(n * NCHUNK, CW)

    z32 = jnp.zeros((ZROWS, CW), jnp.float32)
    ones_in = jnp.ones((SUB, CW), jnp.float32)

    mesh = plsc.VectorSubcoreMesh(core_axis_name="c", subcore_axis_name="s")
    out_sds = jax.ShapeDtypeStruct((NCHUNK, n_pad, CW), jnp.float32)

    @functools.partial(
        pl.kernel,
        out_type=(out_sds, out_sds, out_sds,
                  jax.ShapeDtypeStruct((n_pad, CW), jnp.float32)),
        mesh=mesh,
        compiler_params=pltpu.CompilerParams(use_tc_tiling_on_sc=False),
        scratch_types=dict(
            srcb=pltpu.VMEM((spt, SUB), jnp.int32),
            dstb=pltpu.VMEM((spt, SUB), jnp.int32),
            rows=pltpu.VMEM((NBUF, SUB, CW), jnp.float32),
            onesb=pltpu.VMEM((SUB, CW), jnp.float32),
            acc=pltpu.VMEM_SHARED((n_pad, CW), jnp.float32),
            sems=(pltpu.SemaphoreType.DMA,) * NBUF,
        ),
    )
    def sc_kernel(fp_hbm, fr_hbm, eps_h, epd_h, ecs_h, ecd_h, ets_h, etd_h,
                  z32_h, ones_h,
                  h1_out, h2_out, sq_out, dg_out,
                  srcb, dstb, rows, onesb, acc, sems):
        c = lax.axis_index("c")
        s = lax.axis_index("s")

        pltpu.sync_copy(ones_h, onesb)

        def zero_acc():
            nfull = rpt // ZROWS
            for k in range(nfull):
                pltpu.sync_copy(z32_h,
                                acc.at[pl.ds(s * rpt + k * ZROWS, ZROWS)])
            rem = rpt - nfull * ZROWS
            if rem:
                pltpu.sync_copy(z32_h.at[pl.ds(0, rem)],
                                acc.at[pl.ds(s * rpt + nfull * ZROWS, rem)])

        relations = (
            (fp_hbm, eps_h, epd_h, h1_out),
            (fr_hbm, ecs_h, ecd_h, h2_out),
            (fr_hbm, ets_h, etd_h, sq_out),
        )
        for tbl, src_h, dst_h, out_h in relations:
            for p in range(NCHUNK // NC):
                ch = c + NC * p

                zero_acc()
                plsc.subcore_barrier()

                pltpu.sync_copy(src_h.at[pl.ds(s * spt, spt)], srcb)
                pltpu.sync_copy(dst_h.at[pl.ds(s * spt, spt)], dstb)

                def idx_body(r, _, *, _ch=ch):
                    for j in range(SUB // 16):
                        sl = pl.ds(j * 16, 16)
                        srcb[r, sl] = srcb[r, sl] * NCHUNK + _ch
                    return 0
                lax.fori_loop(0, spt, idx_body, 0)

                for k in range(NBUF):
                    pltpu.async_copy(tbl.at[srcb.at[k]], rows.at[k], sems[k])

                def gs_body(g, _, *, _tbl=tbl):
                    for k in range(NBUF):
                        i = NBUF * g + k
                        pltpu.make_async_copy(_tbl.at[srcb.at[i]],
                                              rows.at[k], sems[k]).wait()
                        pltpu.sync_copy(rows.at[k], acc.at[dstb.at[i]],
                                        add=True)

                        @pl.when(i + NBUF < spt)
                        def _start_next():
                            pltpu.async_copy(_tbl.at[srcb.at[i + NBUF]],
                                             rows.at[k], sems[k])
                    return 0
                lax.fori_loop(0, spt // NBUF, gs_body, 0)
                plsc.subcore_barrier()

                pltpu.sync_copy(
                    acc.at[pl.ds(s * rpt, rpt)],
                    out_h.at[ch, pl.ds(s * rpt, rpt)])

        zero_acc()
        plsc.subcore_barrier()
        pltpu.sync_copy(etd_h.at[pl.ds(s * spt, spt)], dstb)

        def deg_body(i, _):
            pltpu.sync_copy(onesb, acc.at[dstb.at[i]], add=True)
            return 0
        lax.fori_loop(0, spt, deg_body, 0)
        plsc.subcore_barrier()

        @pl.when(c == 0)
        def _deg_out():
            pltpu.sync_copy(acc.at[pl.ds(s * rpt, rpt)],
                            dg_out.at[pl.ds(s * rpt, rpt)])

    return sc_kernel(fp_flat, fr_flat, ep_s, ep_d, ec_s, ec_d, et_s, et_d,
                     z32, ones_in)


def _mlp_body(h1, h2, fr, wr, br, sq, dg, fp, wp, bp, out_r, out_p):
    z = (jnp.dot(h1[...], wr[...][:128, :], preferred_element_type=jnp.float32,
                 precision=lax.Precision.HIGHEST)
         + jnp.dot(h2[...], wr[...][128:, :], preferred_element_type=jnp.float32,
                   precision=lax.Precision.HIGHEST)
         + br[...])
    out_r[...] = fr[...] + jnp.maximum(z, 0.0)
    d = jnp.maximum(dg[...][:, :1], 1.0)
    z2 = (jnp.dot(sq[...] / d, wp[...], preferred_element_type=jnp.float32,
                  precision=lax.Precision.HIGHEST)
          + bp[...])
    out_p[...] = fp[...] + jnp.maximum(z2, 0.0)


def _prep_edges(e, e_pad, n, n_pad):
    src = e[0].astype(jnp.int32)
    dst = e[1].astype(jnp.int32)
    pad = e_pad - src.shape[0]
    if pad:
        r = jnp.arange(pad, dtype=jnp.int32)
        src = jnp.concatenate([src, r % n])
        dst = jnp.concatenate([dst, n + r % (n_pad - n)])
    return (src.reshape(e_pad // SUB, SUB),
            dst.reshape(e_pad // SUB, SUB))


def kernel(feat_router, feat_packet, edge_pass, edge_transfer, edge_connect,
           lin_r_weight, lin_r_bias, lin_p_weight, lin_p_bias):
    n, h = feat_router.shape
    assert feat_packet.shape == (n, h) and h == NCHUNK * CW

    grp = SUB * NS * 8
    e = edge_pass.shape[1]
    e_pad = ((e + grp - 1) // grp) * grp
    n_pad = ((n + 1 + 8 * NS - 1) // (8 * NS)) * (8 * NS)

    ep_s, ep_d = _prep_edges(edge_pass, e_pad, n, n_pad)
    et_s, et_d = _prep_edges(edge_transfer, e_pad, n, n_pad)
    ec_s, ec_d = _prep_edges(edge_connect, e_pad, n, n_pad)

    h1r, h2r, sq, dg = _sc_segment_sums(
        feat_packet, feat_router, ep_s, ep_d, ec_s, ec_d, et_s, et_d, n, h)

    def _rows(x):
        return x.transpose(1, 0, 2).reshape(x.shape[1], h)
    h1r, h2r, sqr = _rows(h1r), _rows(h2r), _rows(sq)

    blk = 1000
    grid = (n // blk,)
    row_spec = pl.BlockSpec((blk, h), lambda i: (i, 0))
    d_spec = pl.BlockSpec((blk, CW), lambda i: (i, 0))
    full = lambda shape: pl.BlockSpec(shape, lambda i: (0, 0))

    out_r, out_p = pl.pallas_call(
        _mlp_body,
        grid=grid,
        in_specs=[row_spec, row_spec, row_spec, full((2 * h, h)),
                  full((1, h)), row_spec, d_spec, row_spec,
                  full((h, h)), full((1, h))],
        out_specs=[row_spec, row_spec],
        out_shape=[jax.ShapeDtypeStruct((n, h), jnp.float32),
                   jax.ShapeDtypeStruct((n, h), jnp.float32)],
    )(h1r, h2r, feat_router, lin_r_weight, lin_r_bias.reshape(1, h),
      sqr, dg, feat_packet, lin_p_weight, lin_p_bias.reshape(1, h))
    return out_r, out_p
